# Initial kernel scaffold; baseline (speedup 1.0000x reference)
import jax
import jax.numpy as jnp
from jax.experimental import pallas as pl


def kernel(input, embedding):
    raise NotImplementedError("write your pallas kernel here")


# speedup vs baseline: 1.4372x; 1.4372x over previous
import functools

import jax
import jax.numpy as jnp
from jax import lax
from jax.experimental import pallas as pl
from jax.experimental.pallas import tpu as pltpu
from jax.experimental.pallas import tpu_sc as plsc

_D = 32
_BATCH = 16384
_FIELDS = 26
_B = _BATCH * _FIELDS
_NC = 2
_NS = 16
_NW = _NC * _NS
_CHUNK = 128
_RPW = _B // _NW
_NCHUNK = _RPW // _CHUNK


def _make_lookup():
    mesh = plsc.VectorSubcoreMesh(core_axis_name="c", subcore_axis_name="s")

    @functools.partial(
        pl.kernel,
        out_type=jax.ShapeDtypeStruct((_B, _D), jnp.float32),
        mesh=mesh,
        scratch_types=[
            pltpu.VMEM((_NCHUNK, _CHUNK), jnp.int32),
            pltpu.VMEM((_CHUNK, _D), jnp.float32),
            pltpu.SemaphoreType.DMA,
        ],
        compiler_params=pltpu.CompilerParams(use_tc_tiling_on_sc=False),
    )
    def lookup(idx_hbm, table_hbm, out_hbm, idx_v, rows_v, sem):
        wid = lax.axis_index("s") * _NC + lax.axis_index("c")
        cbase = wid * _NCHUNK
        pltpu.sync_copy(idx_hbm.at[pl.ds(cbase, _NCHUNK)], idx_v)

        def step(j, carry):
            pltpu.async_copy(table_hbm.at[idx_v.at[j]], rows_v, sem).wait()
            pltpu.sync_copy(rows_v, out_hbm.at[pl.ds((cbase + j) * _CHUNK, _CHUNK)])
            return carry

        lax.fori_loop(0, _NCHUNK, step, 0)

    return lookup


_lookup = _make_lookup()


def kernel(input, embedding):
    idx = input.reshape(_B // _CHUNK, _CHUNK).astype(jnp.int32)
    out = _lookup(idx, embedding)
    return out.reshape(_BATCH, _FIELDS, _D)


# speedup vs baseline: 1.5493x; 1.0780x over previous
import functools

import jax
import jax.numpy as jnp
from jax import lax
from jax.experimental import pallas as pl
from jax.experimental.pallas import tpu as pltpu
from jax.experimental.pallas import tpu_sc as plsc

_D = 32
_BATCH = 16384
_FIELDS = 26
_B = _BATCH * _FIELDS
_NC = 2
_NS = 16
_NW = _NC * _NS
_CHUNK = 128
_RPW = _B // _NW
_NCHUNK = _RPW // _CHUNK
_NBUF = 4
_LAG = 2


def _make_lookup():
    mesh = plsc.VectorSubcoreMesh(core_axis_name="c", subcore_axis_name="s")

    @functools.partial(
        pl.kernel,
        out_type=jax.ShapeDtypeStruct((_B, _D), jnp.float32),
        mesh=mesh,
        scratch_types=[
            pltpu.VMEM((_NCHUNK, _CHUNK), jnp.int32),
            pltpu.VMEM((_NBUF, _CHUNK, _D), jnp.float32),
            pltpu.SemaphoreType.DMA,
            pltpu.SemaphoreType.DMA,
        ],
        compiler_params=pltpu.CompilerParams(use_tc_tiling_on_sc=False),
    )
    def lookup(idx_hbm, table_hbm, out_hbm, idx_v, rows_v, sem_g, sem_s):
        wid = lax.axis_index("s") * _NC + lax.axis_index("c")
        cbase = wid * _NCHUNK
        pltpu.sync_copy(idx_hbm.at[pl.ds(cbase, _NCHUNK)], idx_v)

        def fire_gather(j):
            pltpu.async_copy(
                table_hbm.at[idx_v.at[j]], rows_v.at[lax.rem(j, _NBUF)], sem_g)

        def fire_store(j):
            pltpu.async_copy(
                rows_v.at[lax.rem(j, _NBUF)],
                out_hbm.at[pl.ds((cbase + j) * _CHUNK, _CHUNK)], sem_s)

        def wait_gather():
            pltpu.make_async_copy(
                table_hbm.at[idx_v.at[0]], rows_v.at[0], sem_g).wait()

        def wait_store():
            pltpu.make_async_copy(
                out_hbm.at[pl.ds(cbase * _CHUNK, _CHUNK)], rows_v.at[0],
                sem_s).wait()

        for j in range(_LAG):
            fire_gather(j)

        def step(j, carry):
            wait_gather()
            fire_store(j)
            @pl.when(j >= _LAG)
            def _():
                wait_store()
            @pl.when(j + _LAG < _NCHUNK)
            def _():
                fire_gather(j + _LAG)
            return carry

        lax.fori_loop(0, _NCHUNK, step, 0)

        for _ in range(_LAG):
            wait_store()

    return lookup


_lookup = _make_lookup()


def kernel(input, embedding):
    idx = input.reshape(_B // _CHUNK, _CHUNK).astype(jnp.int32)
    out = _lookup(idx, embedding)
    return out.reshape(_BATCH, _FIELDS, _D)


# speedup vs baseline: 1.5739x; 1.0158x over previous
import functools

import jax
import jax.numpy as jnp
from jax import lax
from jax.experimental import pallas as pl
from jax.experimental.pallas import tpu as pltpu
from jax.experimental.pallas import tpu_sc as plsc

_D = 32
_BATCH = 16384
_FIELDS = 26
_B = _BATCH * _FIELDS
_NC = 2
_NS = 16
_NW = _NC * _NS
_CHUNK = 128
_RPW = _B // _NW
_NCHUNK = _RPW // _CHUNK
_NBUF = 8
_LAG = 4


def _make_lookup():
    mesh = plsc.VectorSubcoreMesh(core_axis_name="c", subcore_axis_name="s")

    @functools.partial(
        pl.kernel,
        out_type=jax.ShapeDtypeStruct((_B, _D), jnp.float32),
        mesh=mesh,
        scratch_types=[
            pltpu.VMEM((_NCHUNK, _CHUNK), jnp.int32),
            pltpu.VMEM((_NBUF, _CHUNK, _D), jnp.float32),
            pltpu.SemaphoreType.DMA,
            pltpu.SemaphoreType.DMA,
        ],
        compiler_params=pltpu.CompilerParams(use_tc_tiling_on_sc=False),
    )
    def lookup(idx_hbm, table_hbm, out_hbm, idx_v, rows_v, sem_g, sem_s):
        wid = lax.axis_index("s") * _NC + lax.axis_index("c")
        cbase = wid * _NCHUNK
        pltpu.sync_copy(idx_hbm.at[pl.ds(cbase, _NCHUNK)], idx_v)

        def fire_gather(j):
            pltpu.async_copy(
                table_hbm.at[idx_v.at[j]], rows_v.at[lax.rem(j, _NBUF)], sem_g)

        def fire_store(j):
            pltpu.async_copy(
                rows_v.at[lax.rem(j, _NBUF)],
                out_hbm.at[pl.ds((cbase + j) * _CHUNK, _CHUNK)], sem_s)

        def wait_gather():
            pltpu.make_async_copy(
                table_hbm.at[idx_v.at[0]], rows_v.at[0], sem_g).wait()

        def wait_store():
            pltpu.make_async_copy(
                out_hbm.at[pl.ds(cbase * _CHUNK, _CHUNK)], rows_v.at[0],
                sem_s).wait()

        for j in range(_LAG):
            fire_gather(j)

        def step(j, carry):
            wait_gather()
            fire_store(j)
            @pl.when(j >= _LAG)
            def _():
                wait_store()
            @pl.when(j + _LAG < _NCHUNK)
            def _():
                fire_gather(j + _LAG)
            return carry

        lax.fori_loop(0, _NCHUNK, step, 0)

        for _ in range(_LAG):
            wait_store()

    return lookup


_lookup = _make_lookup()


def kernel(input, embedding):
    idx = input.reshape(_B // _CHUNK, _CHUNK).astype(jnp.int32)
    out = _lookup(idx, embedding)
    return out.reshape(_BATCH, _FIELDS, _D)


# speedup vs baseline: 1.5804x; 1.0042x over previous
import functools

import jax
import jax.numpy as jnp
from jax import lax
from jax.experimental import pallas as pl
from jax.experimental.pallas import tpu as pltpu
from jax.experimental.pallas import tpu_sc as plsc

_D = 32
_BATCH = 16384
_FIELDS = 26
_B = _BATCH * _FIELDS
_NC = 2
_NS = 16
_NW = _NC * _NS
_CHUNK = 512
_RPW = _B // _NW
_NCHUNK = _RPW // _CHUNK
_NBUF = 4
_LAG = 2


def _make_lookup():
    mesh = plsc.VectorSubcoreMesh(core_axis_name="c", subcore_axis_name="s")

    @functools.partial(
        pl.kernel,
        out_type=jax.ShapeDtypeStruct((_B, _D), jnp.float32),
        mesh=mesh,
        scratch_types=[
            pltpu.VMEM((_NCHUNK, _CHUNK), jnp.int32),
            pltpu.VMEM((_NBUF, _CHUNK, _D), jnp.float32),
            pltpu.SemaphoreType.DMA,
            pltpu.SemaphoreType.DMA,
        ],
        compiler_params=pltpu.CompilerParams(use_tc_tiling_on_sc=False),
    )
    def lookup(idx_hbm, table_hbm, out_hbm, idx_v, rows_v, sem_g, sem_s):
        wid = lax.axis_index("s") * _NC + lax.axis_index("c")
        cbase = wid * _NCHUNK
        pltpu.sync_copy(idx_hbm.at[pl.ds(cbase, _NCHUNK)], idx_v)

        def fire_gather(j):
            pltpu.async_copy(
                table_hbm.at[idx_v.at[j]], rows_v.at[lax.rem(j, _NBUF)], sem_g)

        def fire_store(j):
            pltpu.async_copy(
                rows_v.at[lax.rem(j, _NBUF)],
                out_hbm.at[pl.ds((cbase + j) * _CHUNK, _CHUNK)], sem_s)

        def wait_gather():
            pltpu.make_async_copy(
                table_hbm.at[idx_v.at[0]], rows_v.at[0], sem_g).wait()

        def wait_store():
            pltpu.make_async_copy(
                out_hbm.at[pl.ds(cbase * _CHUNK, _CHUNK)], rows_v.at[0],
                sem_s).wait()

        for j in range(_LAG):
            fire_gather(j)

        def step(j, carry):
            wait_gather()
            fire_store(j)
            @pl.when(j >= _LAG)
            def _():
                wait_store()
            @pl.when(j + _LAG < _NCHUNK)
            def _():
                fire_gather(j + _LAG)
            return carry

        lax.fori_loop(0, _NCHUNK, step, 0)

        for _ in range(_LAG):
            wait_store()

    return lookup


_lookup = _make_lookup()


def kernel(input, embedding):
    idx = input.reshape(_B // _CHUNK, _CHUNK).astype(jnp.int32)
    out = _lookup(idx, embedding)
    return out.reshape(_BATCH, _FIELDS, _D)
